# baseline (speedup 1.0000x reference)
import functools

import jax
import jax.numpy as jnp
from jax import lax
from jax.experimental import pallas as pl
from jax.experimental.pallas import tpu as pltpu
from jax.experimental.pallas import tpu_sc as plsc

_B = 16384
_F = 26
_V = 100000
_D = 16

_L = 16
_NC = 2
_NS = 16
_NW = _NC * _NS
_BPW = _B // _NW
_CB = 128
_NCHUNK = _BPW // _CB
_NIDX = _CB * _F
_NROW = _NIDX // 128
_NVREG = _NIDX // _L


@functools.partial(
    pl.kernel,
    out_type=jax.ShapeDtypeStruct((_B,), jnp.float32),
    mesh=plsc.VectorSubcoreMesh(core_axis_name="c", subcore_axis_name="s"),
    compiler_params=pltpu.CompilerParams(
        needs_layout_passes=False, use_tc_tiling_on_sc=False),
    scratch_types=[
        pltpu.VMEM((_NIDX,), jnp.int32),
        pltpu.VMEM((_NROW, 128), jnp.int32),
        pltpu.VMEM((_NIDX, _D), jnp.float32),
        pltpu.VMEM((_NIDX + _L,), jnp.float32),
        pltpu.VMEM((_BPW,), jnp.float32),
        pltpu.SemaphoreType.DMA,
        pltpu.SemaphoreType.DMA,
    ],
)
def _fm_kernel(idx_hbm, lin_hbm, emb_hbm, out_hbm,
               idx1_v, idx_v, rows_v, lin_v, out_v, sem_e, sem_l):
    cid = lax.axis_index("c")
    sid = lax.axis_index("s")
    wid = sid * _NC + cid
    wbase = wid * _BPW

    def chunk_body(c, carry):
        gbase = pl.multiple_of((wbase + c * _CB) * _F, _NIDX)
        pltpu.sync_copy(idx_hbm.at[pl.ds(gbase, _NIDX)], idx1_v)

        def flat_body(j, carry2):
            r = j // 8
            col = (j % 8) * _L
            p = j * _L + lax.iota(jnp.int32, _L)
            fld = lax.rem(p, _F)
            raw = idx1_v[pl.ds(pl.multiple_of(j * _L, _L), _L)]
            idx_v[r, pl.ds(col, _L)] = raw + fld * _V
            return carry2

        lax.fori_loop(0, _NVREG, flat_body, 0)

        def fire_body(j, carry2):
            off = pl.multiple_of(j * 128, 128)
            pltpu.async_copy(emb_hbm.at[idx_v.at[j]],
                             rows_v.at[pl.ds(off, 128)], sem_e)
            pltpu.async_copy(lin_hbm.at[idx_v.at[j]],
                             lin_v.at[pl.ds(off, 128)], sem_l)
            return carry2

        lax.fori_loop(0, _NROW, fire_body, 0)

        def drain_body(j, carry2):
            off = pl.multiple_of(j * 128, 128)
            pltpu.make_async_copy(emb_hbm.at[idx_v.at[j]],
                                  rows_v.at[pl.ds(off, 128)], sem_e).wait()
            pltpu.make_async_copy(lin_hbm.at[idx_v.at[j]],
                                  lin_v.at[pl.ds(off, 128)], sem_l).wait()
            return carry2

        lax.fori_loop(0, _NROW, drain_body, 0)

        lane = lax.iota(jnp.int32, _L)

        tail_mask = lane < (_F - _L)

        def comp_body(g, carry2):
            vec = jnp.zeros((_L,), jnp.float32)
            for l in range(_L):
                base_i = (g * _L + l) * _F
                s = jnp.zeros((_L,), jnp.float32)
                ss = jnp.zeros((_L,), jnp.float32)
                for f in range(_F):
                    e = rows_v[base_i + f]
                    s = s + e
                    ss = ss + e * e
                lv1 = lin_v[pl.ds(base_i, _L)]
                lv2 = lin_v[pl.ds(base_i + _L, _L)]
                lv2 = jnp.where(tail_mask, lv2, jnp.float32(0.0))
                total = (jnp.float32(0.5) * jnp.sum(s * s - ss)
                         + jnp.sum(lv1) + jnp.sum(lv2))
                vec = jnp.where(lane == l, total, vec)
            out_v[pl.ds(c * _CB + g * _L, _L)] = vec
            return carry2

        lax.fori_loop(0, _CB // _L, comp_body, 0)
        return carry

    lax.fori_loop(0, _NCHUNK, chunk_body, 0)
    pltpu.sync_copy(out_v, out_hbm.at[pl.ds(wbase, _BPW)])


def kernel(indices, linear_table, embed_table, bias):
    idx1 = indices.astype(jnp.int32).reshape(_B * _F)
    lin1 = linear_table.reshape(_F * _V)
    out = _fm_kernel(idx1, lin1, embed_table)
    return bias + out


# speedup vs baseline: 1.0696x; 1.0696x over previous
import functools

import jax
import jax.numpy as jnp
from jax import lax
from jax.experimental import pallas as pl
from jax.experimental.pallas import tpu as pltpu
from jax.experimental.pallas import tpu_sc as plsc

_B = 16384
_F = 26
_V = 100000
_D = 16

_L = 16
_NC = 2
_NS = 16
_NW = _NC * _NS
_BPW = _B // _NW
_CB = 128
_NCHUNK = _BPW // _CB
_NIDX = _CB * _F
_NG = _CB // _L


@functools.partial(
    pl.kernel,
    out_type=jax.ShapeDtypeStruct((_B,), jnp.float32),
    mesh=plsc.VectorSubcoreMesh(core_axis_name="c", subcore_axis_name="s"),
    compiler_params=pltpu.CompilerParams(
        needs_layout_passes=False, use_tc_tiling_on_sc=False),
    scratch_types=[
        pltpu.VMEM((_F, _CB), jnp.int32),
        pltpu.VMEM((_NIDX,), jnp.int32),
        pltpu.VMEM((_D, _NIDX), jnp.float32),
        pltpu.VMEM((_NIDX,), jnp.float32),
        pltpu.VMEM((_BPW,), jnp.float32),
        pltpu.SemaphoreType.DMA,
        pltpu.SemaphoreType.DMA,
    ],
)
def _fm_kernel(idx_hbm, lin_hbm,
               e0, e1, e2, e3, e4, e5, e6, e7,
               e8, e9, e10, e11, e12, e13, e14, e15,
               out_hbm, idx2_v, idxf_v, rowsT_v, lin_v, out_v, sem_e, sem_l):
    embs = (e0, e1, e2, e3, e4, e5, e6, e7,
            e8, e9, e10, e11, e12, e13, e14, e15)
    cid = lax.axis_index("c")
    sid = lax.axis_index("s")
    wid = sid * _NC + cid
    wbase = wid * _BPW

    def chunk_body(c, carry):
        bbase = wbase + c * _CB
        pltpu.sync_copy(idx_hbm.at[:, pl.ds(bbase, _CB)], idx2_v)

        def flat_body(j, carry2):
            f = j // _NG
            g = j - f * _NG
            col = g * _L
            v = idx2_v[f, pl.ds(col, _L)] + f * _V
            idxf_v[pl.ds(pl.multiple_of(f * _CB + col, _L), _L)] = v
            return carry2

        lax.fori_loop(0, _F * _NG, flat_body, 0)

        copies = []
        for d in range(_D):
            copies.append(pltpu.make_async_copy(
                embs[d].at[idxf_v], rowsT_v.at[d], sem_e))
        copies.append(pltpu.make_async_copy(
            lin_hbm.at[idxf_v], lin_v, sem_l))
        for cp in copies:
            cp.start()
        for cp in copies:
            cp.wait()

        def comp_body(g, carry2):
            col = g * _L
            ss = jnp.zeros((_L,), jnp.float32)
            s2 = jnp.zeros((_L,), jnp.float32)
            for d in range(_D):
                s = jnp.zeros((_L,), jnp.float32)
                for f in range(_F):
                    e = rowsT_v[d, pl.ds(f * _CB + col, _L)]
                    s = s + e
                    ss = ss + e * e
                s2 = s2 + s * s
            lin = jnp.zeros((_L,), jnp.float32)
            for f in range(_F):
                lin = lin + lin_v[pl.ds(f * _CB + col, _L)]
            out_v[pl.ds(c * _CB + col, _L)] = (
                jnp.float32(0.5) * (s2 - ss) + lin)
            return carry2

        lax.fori_loop(0, _NG, comp_body, 0)
        return carry

    lax.fori_loop(0, _NCHUNK, chunk_body, 0)
    pltpu.sync_copy(out_v, out_hbm.at[pl.ds(wbase, _BPW)])


def kernel(indices, linear_table, embed_table, bias):
    idx_t = indices.astype(jnp.int32).T
    lin_col = linear_table.T.reshape(_F * _V)
    emb_cols = [embed_table[:, d] for d in range(_D)]
    out = _fm_kernel(idx_t, lin_col, *emb_cols)
    return bias + out
